# baseline (speedup 1.0000x reference)
import functools

import jax
import jax.numpy as jnp
from jax import lax
from jax.experimental import pallas as pl
from jax.experimental.pallas import tpu as pltpu
from jax.experimental.pallas import tpu_sc as plsc

_T, _D, _F, _E = 2048, 768, 3072, 8
_EP1 = _E + 1
_BT = 512
_NB = 17
_NS = _NB * _BT
_NW = 32
_NSG = _NS + _BT
_TPT = _T // _NW
_CC = 64


def _router_body(x_ref, wr_ref, br_ref, w_ref):
    l = lax.dot_general(x_ref[...], wr_ref[...], (((1,), (0,)), ((), ())),
                        preferred_element_type=jnp.float32)
    l = l + br_ref[...]
    lane = lax.broadcasted_iota(jnp.int32, l.shape, 1)
    l = jnp.where(lane < _EP1, l, -jnp.inf)
    m = jnp.max(l, axis=1, keepdims=True)
    p = jnp.exp(l - m)
    p = p / jnp.sum(p, axis=1, keepdims=True)
    m1 = jnp.max(p, axis=1, keepdims=True)
    i1 = jnp.min(jnp.where(p == m1, lane, 10**6), axis=1, keepdims=True)
    p2 = jnp.where(lane == i1, -1.0, p)
    m2 = jnp.max(p2, axis=1, keepdims=True)
    i2 = jnp.min(jnp.where(p2 == m2, lane, 10**6), axis=1, keepdims=True)
    sel = (lane == i1) | (lane == i2)
    w_ref[...] = jnp.where(sel, p, 0.0)


def _router(x, wr, br):
    return pl.pallas_call(
        _router_body,
        grid=(_T // _BT,),
        in_specs=[pl.BlockSpec((_BT, _D), lambda b: (b, 0)),
                  pl.BlockSpec((_D, 128), lambda b: (0, 0)),
                  pl.BlockSpec((1, 128), lambda b: (0, 0))],
        out_specs=pl.BlockSpec((_BT, 128), lambda b: (b, 0)),
        out_shape=jax.ShapeDtypeStruct((_T, 128), jnp.float32),
    )(x, wr, br)


def _mlp_body(info_ref, xg_ref, wfc_ref, bfc_ref, wpj_ref, bpj_ref, o_ref):
    b = pl.program_id(0)
    act = info_ref[0, _NB]
    be = info_ref[0, b]

    @pl.when(jnp.logical_and(b < act, be == _E))
    def _():
        o_ref[...] = xg_ref[...]

    @pl.when(jnp.logical_and(b < act, be < _E))
    def _():
        xb = xg_ref[...].astype(jnp.bfloat16)
        h = lax.dot_general(xb, wfc_ref[0].astype(jnp.bfloat16),
                            (((1,), (0,)), ((), ())),
                            preferred_element_type=jnp.float32)
        h = h + bfc_ref[0]
        g = 0.5 * h * (1.0 + lax.erf(h * (2.0 ** -0.5)))
        o = lax.dot_general(g.astype(jnp.bfloat16),
                            wpj_ref[0].astype(jnp.bfloat16),
                            (((1,), (0,)), ((), ())),
                            preferred_element_type=jnp.float32)
        o_ref[...] = o + bpj_ref[0]


def _mlp(info, xg, wfc_bf, bfc, wpj_bf, bpj):
    grid_spec = pltpu.PrefetchScalarGridSpec(
        num_scalar_prefetch=1,
        grid=(_NB,),
        in_specs=[
            pl.BlockSpec((_BT, _D),
                         lambda b, info: (jnp.minimum(b, info[0, _NB] - 1),
                                          0)),
            pl.BlockSpec((1, _D, _F),
                         lambda b, info: (jnp.minimum(info[0, b], _E - 1), 0, 0)),
            pl.BlockSpec((1, 1, _F),
                         lambda b, info: (jnp.minimum(info[0, b], _E - 1), 0, 0)),
            pl.BlockSpec((1, _F, _D),
                         lambda b, info: (jnp.minimum(info[0, b], _E - 1), 0, 0)),
            pl.BlockSpec((1, 1, _D),
                         lambda b, info: (jnp.minimum(info[0, b], _E - 1), 0, 0)),
        ],
        out_specs=pl.BlockSpec((_BT, _D), lambda b, info: (b, 0)),
    )
    return pl.pallas_call(
        _mlp_body,
        grid_spec=grid_spec,
        out_shape=jax.ShapeDtypeStruct((_NS, _D), jnp.float32),
        compiler_params=pltpu.CompilerParams(
            dimension_semantics=("arbitrary",)),
    )(info, xg, wfc_bf, bfc, wpj_bf, bpj)


def _plumb_body(x_ref, wr_ref, br_ref, d0_ref, d1_ref, sa_ref, sb_ref,
                wa_ref, wb_ref, info_ref):
    lane = lax.broadcasted_iota(jnp.int32, (_T, 128), 1)
    l = lax.dot_general(x_ref[...], wr_ref[...], (((1,), (0,)), ((), ())),
                        preferred_element_type=jnp.float32)
    l = l + br_ref[...]
    l = jnp.where(lane < _EP1, l, -jnp.inf)
    m = jnp.max(l, axis=1, keepdims=True)
    p = jnp.exp(l - m)
    p = p / jnp.sum(p, axis=1, keepdims=True)
    m1 = jnp.max(p, axis=1, keepdims=True)
    i1 = jnp.min(jnp.where(p == m1, lane, 10**6), axis=1, keepdims=True)
    p2 = jnp.where(lane == i1, -1.0, p)
    m2 = jnp.max(p2, axis=1, keepdims=True)
    i2 = jnp.min(jnp.where(p2 == m2, lane, 10**6), axis=1, keepdims=True)
    sel = (lane == i1) | (lane == i2)
    w = jnp.where(sel, p, 0.0)
    mask = w > 0.0
    mf = mask.astype(jnp.float32)
    mb = mask.astype(jnp.bfloat16)
    r5 = lax.broadcasted_iota(jnp.int32, (_BT, _BT), 0)
    c5 = lax.broadcasted_iota(jnp.int32, (_BT, _BT), 1)
    tril = (c5 < r5).astype(jnp.bfloat16)
    mb3 = mb.reshape(_T // _BT, _BT, 128)
    mf3 = mf.reshape(_T // _BT, _BT, 128)
    ranks = []
    offs = jnp.zeros((1, 128), jnp.float32)
    for blk in range(_T // _BT):
        rb = lax.dot_general(tril, mb3[blk], (((1,), (0,)), ((), ())),
                             preferred_element_type=jnp.float32)
        ranks.append(rb + offs)
        offs = offs + jnp.sum(mf3[blk], axis=0, keepdims=True)
    rank = jnp.concatenate(ranks, axis=0)
    cnt = offs
    lane1 = lax.broadcasted_iota(jnp.int32, (1, 128), 1)
    nblk = jnp.floor((cnt + (_BT - 1)) * (1.0 / _BT))
    nblk = jnp.where(lane1 < _EP1, nblk, 0.0)
    ur = lax.broadcasted_iota(jnp.int32, (128, 128), 0)
    uc = lax.broadcasted_iota(jnp.int32, (128, 128), 1)
    triu = (ur <= uc).astype(jnp.bfloat16)
    incl = lax.dot_general(nblk.astype(jnp.bfloat16), triu,
                           (((1,), (0,)), ((), ())),
                           preferred_element_type=jnp.float32)
    act = jnp.sum(jnp.where(lane1 == _E, incl, 0.0), axis=1, keepdims=True)
    slot_base = (incl - nblk) * float(_BT)
    slotf = slot_base + rank
    nsel = jnp.sum(mf, axis=1, keepdims=True)
    s_min = jnp.min(jnp.where(mask, slotf, 1e9), axis=1, keepdims=True)
    s_max = jnp.max(jnp.where(mask, slotf, -1.0), axis=1, keepdims=True)
    row1 = lax.broadcasted_iota(jnp.int32, (_T, 1), 0)
    trash = (_NS + (row1 & (_BT - 1))).astype(jnp.float32)
    d0_ref[...] = jnp.where(nsel >= 1.0, s_min, trash).astype(jnp.int32)
    d1_ref[...] = jnp.where(nsel >= 2.0, s_max, trash).astype(jnp.int32)
    sa_ref[...] = jnp.where(nsel >= 1.0, s_min, 0.0).astype(jnp.int32)
    sb_ref[...] = jnp.where(nsel >= 2.0, s_max, 0.0).astype(jnp.int32)
    fe = jnp.min(jnp.where(mask, lane, 99), axis=1, keepdims=True)
    le = jnp.max(jnp.where(mask, lane, -1), axis=1, keepdims=True)
    wa = jnp.sum(jnp.where(lane == fe, w, 0.0), axis=1, keepdims=True)
    wb = jnp.where(nsel >= 2.0,
                   jnp.sum(jnp.where(lane == le, w, 0.0), axis=1,
                           keepdims=True), 0.0)
    wa_ref[...] = jnp.broadcast_to(wa, (_T, 16))
    wb_ref[...] = jnp.broadcast_to(wb, (_T, 16))
    lane1f = lane1.astype(jnp.float32)
    be_raw = jnp.zeros((1, 128), jnp.float32)
    for e in range(_EP1):
        incl_e = jnp.sum(jnp.where(lane1 == e, incl, 0.0), axis=1,
                         keepdims=True)
        be_raw = be_raw + (lane1f >= incl_e).astype(jnp.float32)
    last_be = jnp.sum(jnp.where(lane1f == act - 1.0, be_raw, 0.0), axis=1,
                      keepdims=True)
    bev = jnp.where(lane1f < act, be_raw, last_be)
    info = jnp.where(lane1 == _NB, act, bev)
    info_ref[...] = info.astype(jnp.int32)


def _plumb(x, wr, br):
    return pl.pallas_call(
        _plumb_body,
        out_shape=[jax.ShapeDtypeStruct((_T, 1), jnp.int32),
                   jax.ShapeDtypeStruct((_T, 1), jnp.int32),
                   jax.ShapeDtypeStruct((_T, 1), jnp.int32),
                   jax.ShapeDtypeStruct((_T, 1), jnp.int32),
                   jax.ShapeDtypeStruct((_T, 16), jnp.float32),
                   jax.ShapeDtypeStruct((_T, 16), jnp.float32),
                   jax.ShapeDtypeStruct((1, 128), jnp.int32)],
    )(x, wr, br)


def _build_dispatch():
    mesh = plsc.VectorSubcoreMesh(core_axis_name="c", subcore_axis_name="s")

    @functools.partial(
        pl.kernel, mesh=mesh,
        out_type=jax.ShapeDtypeStruct((_NSG, _D), jnp.float32),
        scratch_types=[pltpu.VMEM((_TPT,), jnp.int32),
                       pltpu.VMEM((_TPT,), jnp.int32),
                       pltpu.VMEM((_TPT, _D), jnp.float32),
                       pltpu.SemaphoreType.DMA,
                       pltpu.SemaphoreType.DMA],
    )
    def dk(x_hbm, d0_hbm, d1_hbm, out_hbm, i0_v, i1_v, rows_v, sem0, sem1):
        wid = lax.axis_index("s") * 2 + lax.axis_index("c")
        base = wid * _TPT
        pltpu.sync_copy(d0_hbm.at[pl.ds(base, _TPT)], i0_v)
        pltpu.sync_copy(d1_hbm.at[pl.ds(base, _TPT)], i1_v)
        pltpu.sync_copy(x_hbm.at[pl.ds(base, _TPT)], rows_v)
        c0 = pltpu.async_copy(rows_v, out_hbm.at[i0_v], sem0)
        c1 = pltpu.async_copy(rows_v, out_hbm.at[i1_v], sem1)
        c0.wait()
        c1.wait()

    return dk


def _build_combine():
    mesh = plsc.VectorSubcoreMesh(core_axis_name="c", subcore_axis_name="s")

    nch = _TPT // _CC

    @functools.partial(
        pl.kernel, mesh=mesh,
        out_type=jax.ShapeDtypeStruct((_T, _D), jnp.float32),
        scratch_types=[[pltpu.VMEM((_CC,), jnp.int32) for _ in range(2 * nch)],
                       [pltpu.VMEM((_CC, 16), jnp.float32)
                        for _ in range(2 * nch)],
                       [pltpu.VMEM((_CC, _D), jnp.float32)
                        for _ in range(2 * nch)],
                       [pltpu.SemaphoreType.DMA for _ in range(2 * nch)],
                       pltpu.SemaphoreType.DMA],
    )
    def ck(og_hbm, sa_hbm, sb_hbm, wa_hbm, wb_hbm, out_hbm,
           idxs, wgt, bufs, gsems, ssem):
        wid = lax.axis_index("s") * 2 + lax.axis_index("c")
        base = wid * _TPT
        stg = []
        for c in range(nch):
            off = base + c * _CC
            stg.append(pltpu.async_copy(sa_hbm.at[pl.ds(off, _CC)],
                                        idxs[2 * c], gsems[2 * c]))
            stg.append(pltpu.async_copy(sb_hbm.at[pl.ds(off, _CC)],
                                        idxs[2 * c + 1], gsems[2 * c + 1]))
            stg.append(pltpu.async_copy(wa_hbm.at[pl.ds(off, _CC)],
                                        wgt[2 * c], ssem))
            stg.append(pltpu.async_copy(wb_hbm.at[pl.ds(off, _CC)],
                                        wgt[2 * c + 1], ssem))
        for cp in stg:
            cp.wait()
        cps = [pltpu.async_copy(og_hbm.at[idxs[k]], bufs[k], gsems[k])
               for k in range(2 * nch)]
        scps = []
        for c in range(nch):
            a_v, b_v = bufs[2 * c], bufs[2 * c + 1]
            wa_v, wb_v = wgt[2 * c], wgt[2 * c + 1]
            cps[2 * c].wait()
            cps[2 * c + 1].wait()

            @plsc.parallel_loop(0, _CC, 1, unroll=4)
            def _row(r):
                wa = wa_v[r, pl.ds(0, 16)]
                wb = wb_v[r, pl.ds(0, 16)]
                for k in range(_D // 16):
                    s = pl.ds(k * 16, 16)
                    a_v[r, s] = a_v[r, s] * wa + b_v[r, s] * wb

            scps.append(
                pltpu.async_copy(a_v, out_hbm.at[pl.ds(base + c * _CC, _CC)],
                                 ssem))
        for s in scps:
            s.wait()

    return ck


_sc_cache = {}


def _dispatch_rows(x, d0, d1):
    if "d" not in _sc_cache:
        _sc_cache["d"] = _build_dispatch()
    return _sc_cache["d"](x, d0, d1)


def _combine_rows(og, slot_a, slot_b, wa, wb):
    if "c" not in _sc_cache:
        _sc_cache["c"] = _build_combine()
    return _sc_cache["c"](og, slot_a, slot_b, wa, wb)


def kernel(x, W_router, b_router, W_fc, b_fc, W_proj, b_proj):
    wr = jnp.zeros((_D, 128), jnp.float32).at[:, :_EP1].set(W_router)
    br = jnp.zeros((1, 128), jnp.float32).at[0, :_EP1].set(b_router)
    d0, d1, sa, sb, wa16, wb16, info = _plumb(x, wr, br)
    xg = _dispatch_rows(x, d0.reshape(_T), d1.reshape(_T))
    og = _mlp(info, xg, W_fc, b_fc.reshape(_E, 1, _F),
              W_proj, b_proj.reshape(_E, 1, _D))
    out = _combine_rows(og, sa.reshape(_T), sb.reshape(_T), wa16, wb16)
    return out


# speedup vs baseline: 1.0047x; 1.0047x over previous
import functools

import jax
import jax.numpy as jnp
from jax import lax
from jax.experimental import pallas as pl
from jax.experimental.pallas import tpu as pltpu
from jax.experimental.pallas import tpu_sc as plsc

_T, _D, _F, _E = 2048, 768, 3072, 8
_EP1 = _E + 1
_BT = 512
_NB = 17
_NS = _NB * _BT
_NW = 32
_NSG = _NS + _BT
_TPT = _T // _NW
_CC = 32


def _mlp_body(info_ref, xg_ref, wfc_ref, bfc_ref, wpj_ref, bpj_ref, o_ref):
    b = pl.program_id(0)
    act = info_ref[0, _NB]
    be = info_ref[0, b]

    @pl.when(jnp.logical_and(b < act, be == _E))
    def _():
        o_ref[...] = xg_ref[...]

    @pl.when(jnp.logical_and(b < act, be < _E))
    def _():
        xb = xg_ref[...].astype(jnp.bfloat16)
        h = lax.dot_general(xb, wfc_ref[0].astype(jnp.bfloat16),
                            (((1,), (0,)), ((), ())),
                            preferred_element_type=jnp.float32)
        h = h + bfc_ref[0]
        g = 0.5 * h * (1.0 + lax.erf(h * (2.0 ** -0.5)))
        o = lax.dot_general(g.astype(jnp.bfloat16),
                            wpj_ref[0].astype(jnp.bfloat16),
                            (((1,), (0,)), ((), ())),
                            preferred_element_type=jnp.float32)
        o_ref[...] = o + bpj_ref[0]


def _mlp(info, xg, wfc_bf, bfc, wpj_bf, bpj):
    grid_spec = pltpu.PrefetchScalarGridSpec(
        num_scalar_prefetch=1,
        grid=(_NB,),
        in_specs=[
            pl.BlockSpec((_BT, _D),
                         lambda b, info: (jnp.minimum(b, info[0, _NB] - 1),
                                          0)),
            pl.BlockSpec((1, _D, _F),
                         lambda b, info: (jnp.minimum(info[0, b], _E - 1), 0, 0)),
            pl.BlockSpec((1, 1, _F),
                         lambda b, info: (jnp.minimum(info[0, b], _E - 1), 0, 0)),
            pl.BlockSpec((1, _F, _D),
                         lambda b, info: (jnp.minimum(info[0, b], _E - 1), 0, 0)),
            pl.BlockSpec((1, 1, _D),
                         lambda b, info: (jnp.minimum(info[0, b], _E - 1), 0, 0)),
        ],
        out_specs=pl.BlockSpec((_BT, _D), lambda b, info: (b, 0)),
    )
    return pl.pallas_call(
        _mlp_body,
        grid_spec=grid_spec,
        out_shape=jax.ShapeDtypeStruct((_NS, _D), jnp.float32),
        compiler_params=pltpu.CompilerParams(
            dimension_semantics=("arbitrary",)),
    )(info, xg, wfc_bf, bfc, wpj_bf, bpj)


def _plumb_body(x_ref, wr_ref, br_ref, d0_ref, d1_ref, sa_ref, sb_ref,
                wa_ref, wb_ref, info_ref):
    lane = lax.broadcasted_iota(jnp.int32, (_T, 128), 1)
    l = lax.dot_general(x_ref[...], wr_ref[...], (((1,), (0,)), ((), ())),
                        preferred_element_type=jnp.float32)
    l = l + br_ref[...]
    l = jnp.where(lane < _EP1, l, -jnp.inf)
    m = jnp.max(l, axis=1, keepdims=True)
    p = jnp.exp(l - m)
    p = p / jnp.sum(p, axis=1, keepdims=True)
    m1 = jnp.max(p, axis=1, keepdims=True)
    i1 = jnp.min(jnp.where(p == m1, lane, 10**6), axis=1, keepdims=True)
    p2 = jnp.where(lane == i1, -1.0, p)
    m2 = jnp.max(p2, axis=1, keepdims=True)
    i2 = jnp.min(jnp.where(p2 == m2, lane, 10**6), axis=1, keepdims=True)
    sel = (lane == i1) | (lane == i2)
    w = jnp.where(sel, p, 0.0)
    mask = w > 0.0
    mf = mask.astype(jnp.float32)
    mb = mask.astype(jnp.bfloat16)
    r5 = lax.broadcasted_iota(jnp.int32, (_BT, _BT), 0)
    c5 = lax.broadcasted_iota(jnp.int32, (_BT, _BT), 1)
    tril = (c5 < r5).astype(jnp.bfloat16)
    mb3 = mb.reshape(_T // _BT, _BT, 128)
    mf3 = mf.reshape(_T // _BT, _BT, 128)
    ranks = []
    offs = jnp.zeros((1, 128), jnp.float32)
    for blk in range(_T // _BT):
        rb = lax.dot_general(tril, mb3[blk], (((1,), (0,)), ((), ())),
                             preferred_element_type=jnp.float32)
        ranks.append(rb + offs)
        offs = offs + jnp.sum(mf3[blk], axis=0, keepdims=True)
    rank = jnp.concatenate(ranks, axis=0)
    cnt = offs
    lane1 = lax.broadcasted_iota(jnp.int32, (1, 128), 1)
    nblk = jnp.floor((cnt + (_BT - 1)) * (1.0 / _BT))
    nblk = jnp.where(lane1 < _EP1, nblk, 0.0)
    ur = lax.broadcasted_iota(jnp.int32, (128, 128), 0)
    uc = lax.broadcasted_iota(jnp.int32, (128, 128), 1)
    triu = (ur <= uc).astype(jnp.bfloat16)
    incl = lax.dot_general(nblk.astype(jnp.bfloat16), triu,
                           (((1,), (0,)), ((), ())),
                           preferred_element_type=jnp.float32)
    act = jnp.sum(jnp.where(lane1 == _E, incl, 0.0), axis=1, keepdims=True)
    slot_base = (incl - nblk) * float(_BT)
    slotf = slot_base + rank
    nsel = jnp.sum(mf, axis=1, keepdims=True)
    s_min = jnp.min(jnp.where(mask, slotf, 1e9), axis=1, keepdims=True)
    s_max = jnp.max(jnp.where(mask, slotf, -1.0), axis=1, keepdims=True)
    row1 = lax.broadcasted_iota(jnp.int32, (_T, 1), 0)
    trash = (_NS + (row1 & (_BT - 1))).astype(jnp.float32)
    d0_ref[...] = jnp.where(nsel >= 1.0, s_min, trash).astype(jnp.int32)
    d1_ref[...] = jnp.where(nsel >= 2.0, s_max, trash).astype(jnp.int32)
    sa_ref[...] = jnp.where(nsel >= 1.0, s_min, 0.0).astype(jnp.int32)
    sb_ref[...] = jnp.where(nsel >= 2.0, s_max, 0.0).astype(jnp.int32)
    fe = jnp.min(jnp.where(mask, lane, 99), axis=1, keepdims=True)
    le = jnp.max(jnp.where(mask, lane, -1), axis=1, keepdims=True)
    wa = jnp.sum(jnp.where(lane == fe, w, 0.0), axis=1, keepdims=True)
    wb = jnp.where(nsel >= 2.0,
                   jnp.sum(jnp.where(lane == le, w, 0.0), axis=1,
                           keepdims=True), 0.0)
    wa_ref[...] = jnp.broadcast_to(wa, (_T, 16))
    wb_ref[...] = jnp.broadcast_to(wb, (_T, 16))
    lane1f = lane1.astype(jnp.float32)
    be_raw = jnp.zeros((1, 128), jnp.float32)
    for e in range(_EP1):
        incl_e = jnp.sum(jnp.where(lane1 == e, incl, 0.0), axis=1,
                         keepdims=True)
        be_raw = be_raw + (lane1f >= incl_e).astype(jnp.float32)
    last_be = jnp.sum(jnp.where(lane1f == act - 1.0, be_raw, 0.0), axis=1,
                      keepdims=True)
    bev = jnp.where(lane1f < act, be_raw, last_be)
    info = jnp.where(lane1 == _NB, act, bev)
    info_ref[...] = info.astype(jnp.int32)


def _plumb(x, wr, br):
    return pl.pallas_call(
        _plumb_body,
        out_shape=[jax.ShapeDtypeStruct((_T, 1), jnp.int32),
                   jax.ShapeDtypeStruct((_T, 1), jnp.int32),
                   jax.ShapeDtypeStruct((_T, 1), jnp.int32),
                   jax.ShapeDtypeStruct((_T, 1), jnp.int32),
                   jax.ShapeDtypeStruct((_T, 16), jnp.float32),
                   jax.ShapeDtypeStruct((_T, 16), jnp.float32),
                   jax.ShapeDtypeStruct((1, 128), jnp.int32)],
    )(x, wr, br)


def _build_dispatch():
    mesh = plsc.VectorSubcoreMesh(core_axis_name="c", subcore_axis_name="s")

    @functools.partial(
        pl.kernel, mesh=mesh,
        out_type=jax.ShapeDtypeStruct((_NSG, _D), jnp.float32),
        scratch_types=[pltpu.VMEM((_TPT,), jnp.int32),
                       pltpu.VMEM((_TPT,), jnp.int32),
                       pltpu.VMEM((_TPT, _D), jnp.float32),
                       pltpu.SemaphoreType.DMA,
                       pltpu.SemaphoreType.DMA],
    )
    def dk(x_hbm, d0_hbm, d1_hbm, out_hbm, i0_v, i1_v, rows_v, sem0, sem1):
        wid = lax.axis_index("s") * 2 + lax.axis_index("c")
        base = wid * _TPT
        pltpu.sync_copy(d0_hbm.at[pl.ds(base, _TPT)], i0_v)
        pltpu.sync_copy(d1_hbm.at[pl.ds(base, _TPT)], i1_v)
        pltpu.sync_copy(x_hbm.at[pl.ds(base, _TPT)], rows_v)
        c0 = pltpu.async_copy(rows_v, out_hbm.at[i0_v], sem0)
        c1 = pltpu.async_copy(rows_v, out_hbm.at[i1_v], sem1)
        c0.wait()
        c1.wait()

    return dk


def _build_combine():
    mesh = plsc.VectorSubcoreMesh(core_axis_name="c", subcore_axis_name="s")

    nch = _TPT // _CC

    @functools.partial(
        pl.kernel, mesh=mesh,
        out_type=jax.ShapeDtypeStruct((_T, _D), jnp.float32),
        scratch_types=[[pltpu.VMEM((_CC,), jnp.int32) for _ in range(2 * nch)],
                       [pltpu.VMEM((_CC, 16), jnp.float32)
                        for _ in range(2 * nch)],
                       [pltpu.VMEM((_CC, _D), jnp.float32)
                        for _ in range(2 * nch)],
                       [pltpu.SemaphoreType.DMA for _ in range(2 * nch)],
                       pltpu.SemaphoreType.DMA],
    )
    def ck(og_hbm, sa_hbm, sb_hbm, wa_hbm, wb_hbm, out_hbm,
           idxs, wgt, bufs, gsems, ssem):
        wid = lax.axis_index("s") * 2 + lax.axis_index("c")
        base = wid * _TPT
        stg = []
        for c in range(nch):
            off = base + c * _CC
            stg.append(pltpu.async_copy(sa_hbm.at[pl.ds(off, _CC)],
                                        idxs[2 * c], gsems[2 * c]))
            stg.append(pltpu.async_copy(sb_hbm.at[pl.ds(off, _CC)],
                                        idxs[2 * c + 1], gsems[2 * c + 1]))
            stg.append(pltpu.async_copy(wa_hbm.at[pl.ds(off, _CC)],
                                        wgt[2 * c], ssem))
            stg.append(pltpu.async_copy(wb_hbm.at[pl.ds(off, _CC)],
                                        wgt[2 * c + 1], ssem))
        for cp in stg:
            cp.wait()
        cps = [pltpu.async_copy(og_hbm.at[idxs[k]], bufs[k], gsems[k])
               for k in range(2 * nch)]
        scps = []
        for c in range(nch):
            a_v, b_v = bufs[2 * c], bufs[2 * c + 1]
            wa_v, wb_v = wgt[2 * c], wgt[2 * c + 1]
            cps[2 * c].wait()
            cps[2 * c + 1].wait()

            @plsc.parallel_loop(0, _CC, 1, unroll=4)
            def _row(r):
                wa = wa_v[r, pl.ds(0, 16)]
                wb = wb_v[r, pl.ds(0, 16)]
                for k in range(_D // 16):
                    s = pl.ds(k * 16, 16)
                    a_v[r, s] = a_v[r, s] * wa + b_v[r, s] * wb

            scps.append(
                pltpu.async_copy(a_v, out_hbm.at[pl.ds(base + c * _CC, _CC)],
                                 ssem))
        for s in scps:
            s.wait()

    return ck


_sc_cache = {}


def _dispatch_rows(x, d0, d1):
    if "d" not in _sc_cache:
        _sc_cache["d"] = _build_dispatch()
    return _sc_cache["d"](x, d0, d1)


def _combine_rows(og, slot_a, slot_b, wa, wb):
    if "c" not in _sc_cache:
        _sc_cache["c"] = _build_combine()
    return _sc_cache["c"](og, slot_a, slot_b, wa, wb)


def kernel(x, W_router, b_router, W_fc, b_fc, W_proj, b_proj):
    wr = jnp.zeros((_D, 128), jnp.float32).at[:, :_EP1].set(W_router)
    br = jnp.zeros((1, 128), jnp.float32).at[0, :_EP1].set(b_router)
    d0, d1, sa, sb, wa16, wb16, info = _plumb(x, wr, br)
    xg = _dispatch_rows(x, d0.reshape(_T), d1.reshape(_T))
    og = _mlp(info, xg, W_fc, b_fc.reshape(_E, 1, _F),
              W_proj, b_proj.reshape(_E, 1, _D))
    out = _combine_rows(og, sa.reshape(_T), sb.reshape(_T), wa16, wb16)
    return out
